# baseline (speedup 1.0000x reference)
import functools

import jax
import jax.numpy as jnp
from jax import lax
from jax.experimental import pallas as pl
from jax.experimental.pallas import tpu as pltpu
from jax.experimental.pallas import tpu_sc as plsc

N_ROWS = 8192
D = 256
K = 8192
BN = 1024
BK = 2048
LANES = 128
RG = 64
N_TILES = N_ROWS // BN
K_TILES = K // BK
N_CHUNKS = BK // LANES


def _dist_argmin_body(z_ref, cbt_ref, zn_ref, cn_ref, idx_ref, loss_ref,
                      acc_ref, rv_ref, ri_ref):
    n = pl.program_id(0)
    k = pl.program_id(1)
    zb = z_ref[...]
    cbt = cbt_ref[...]
    znb = zn_ref[...]
    cnb = cn_ref[...]
    s = lax.dot_general(zb, cbt, (((1,), (0,)), ((), ())),
                        preferred_element_type=jnp.float32)
    zcol = znb[:, None]

    @pl.when(k == 0)
    def _():
        rv_ref[...] = jnp.full((BN, LANES), jnp.inf, jnp.float32)
        ri_ref[...] = jnp.zeros((BN, LANES), jnp.int32)

    for rg in range(BN // RG):
        rsl = slice(rg * RG, (rg + 1) * RG)
        rds = pl.ds(rg * RG, RG)
        rv = rv_ref[rds, :]
        ri = ri_ref[rds, :]
        zc = zcol[rsl]
        for c in range(N_CHUNKS):
            sl = slice(c * LANES, (c + 1) * LANES)
            dj = (zc + s[rsl, sl]) + cnb[None, sl]
            upd = dj < rv
            rv = jnp.where(upd, dj, rv)
            ri = jnp.where(upd, k * N_CHUNKS + c, ri)
        rv_ref[rds, :] = rv
        ri_ref[rds, :] = ri

    @pl.when(k == K_TILES - 1)
    def _():
        lane = lax.broadcasted_iota(jnp.int32, (BN, LANES), 1)
        rva = rv_ref[...]
        ria = ri_ref[...]
        tmin = jnp.min(rva, axis=1)
        idx_ref[...] = jnp.min(
            jnp.where(rva <= tmin[:, None], ria * LANES + lane, 2**30), axis=1)
        part = jnp.sum(tmin)

        @pl.when(n == 0)
        def _():
            acc_ref[0, 0] = part

        @pl.when(n > 0)
        def _():
            acc_ref[0, 0] = acc_ref[0, 0] + part

        @pl.when(n == N_TILES - 1)
        def _():
            loss_ref[0, 0] = acc_ref[0, 0] * (1.25 / (N_ROWS * D))


def _dist_argmin(zf, cbt, zn, cn):
    return pl.pallas_call(
        _dist_argmin_body,
        grid=(N_TILES, K_TILES),
        in_specs=[
            pl.BlockSpec((BN, D), lambda n, k: (n, 0)),
            pl.BlockSpec((D, BK), lambda n, k: (0, k)),
            pl.BlockSpec((BN,), lambda n, k: (n,)),
            pl.BlockSpec((BK,), lambda n, k: (k,)),
        ],
        out_specs=[
            pl.BlockSpec((BN,), lambda n, k: (n,)),
            pl.BlockSpec(memory_space=pltpu.SMEM),
        ],
        out_shape=[
            jax.ShapeDtypeStruct((N_ROWS,), jnp.int32),
            jax.ShapeDtypeStruct((1, 1), jnp.float32),
        ],
        scratch_shapes=[
            pltpu.SMEM((1, 1), jnp.float32),
            pltpu.VMEM((BN, LANES), jnp.float32),
            pltpu.VMEM((BN, LANES), jnp.int32),
        ],
        compiler_params=pltpu.CompilerParams(
            dimension_semantics=("arbitrary", "arbitrary")),
    )(zf, cbt, zn, cn)


@functools.cache
def _make_sc_gather():
    info = plsc.get_sparse_core_info()
    nc, ns = info.num_cores, info.num_subcores
    nw = nc * ns
    b_per_w = N_ROWS // nw
    n_chunks = b_per_w // 128
    mesh = plsc.VectorSubcoreMesh(core_axis_name="c", subcore_axis_name="s")

    @functools.partial(
        pl.kernel, mesh=mesh,
        out_type=jax.ShapeDtypeStruct((N_ROWS, D), jnp.float32),
        scratch_types=[
            pltpu.VMEM((n_chunks, 128), jnp.int32),
            pltpu.VMEM((b_per_w, D), jnp.float32),
            pltpu.SemaphoreType.DMA,
        ],
    )
    def gather(idx_hbm, table_hbm, out_hbm, idx_v, rows_v, sem):
        wid = lax.axis_index("s") * nc + lax.axis_index("c")
        base = wid * n_chunks
        pltpu.sync_copy(idx_hbm.at[pl.ds(base, n_chunks)], idx_v)
        handles = [
            pltpu.async_copy(table_hbm.at[idx_v.at[j]],
                             rows_v.at[pl.ds(j * 128, 128)], sem)
            for j in range(n_chunks)
        ]
        for h in handles:
            h.wait()
        pltpu.sync_copy(rows_v, out_hbm.at[pl.ds(wid * b_per_w, b_per_w)])

    return gather


def kernel(x, codebook):
    zf = jnp.transpose(x, (0, 2, 3, 1)).reshape(N_ROWS, D)
    zn = jnp.sum(zf ** 2, axis=1)
    cn = jnp.sum(codebook ** 2, axis=1)
    idx, loss2 = _dist_argmin(zf, codebook.T * -2.0, zn, cn)
    quant = _make_sc_gather()(idx.reshape(N_ROWS // 128, 128), codebook)
    out = quant.reshape(8, 32, 32, D).transpose(0, 3, 1, 2)
    return out, loss2[0, 0], idx


# speedup vs baseline: 1.2383x; 1.2383x over previous
import functools

import jax
import jax.numpy as jnp
from jax import lax
from jax.experimental import pallas as pl
from jax.experimental.pallas import tpu as pltpu
from jax.experimental.pallas import tpu_sc as plsc

N_ROWS = 8192
D = 256
K = 8192
BN = 1024
BK = 2048
LANES = 128
RG = 64
QCHUNKS = 4
N_TILES = N_ROWS // BN
K_TILES = K // BK
N_CHUNKS = BK // LANES


def _dist_argmin_body(z_ref, cbt_ref, zn_ref, cn_ref, idx_ref, loss_ref,
                      acc_ref, rv_ref, ri_ref):
    n = pl.program_id(0)
    k = pl.program_id(1)
    zb = z_ref[...]
    cbt = cbt_ref[...]
    znb = zn_ref[...]
    cnb = cn_ref[...]
    zcol = znb[:, None]

    @pl.when(k == 0)
    def _():
        rv_ref[...] = jnp.full((BN, LANES), jnp.inf, jnp.float32)
        ri_ref[...] = jnp.zeros((BN, LANES), jnp.int32)

    nq = N_CHUNKS // QCHUNKS
    for q in range(nq):
        qsl = slice(q * QCHUNKS * LANES, (q + 1) * QCHUNKS * LANES)
        sq = lax.dot_general(zb, cbt[:, qsl], (((1,), (0,)), ((), ())),
                             preferred_element_type=jnp.float32)
        for rg in range(BN // RG):
            rsl = slice(rg * RG, (rg + 1) * RG)
            rds = pl.ds(rg * RG, RG)
            rv = rv_ref[rds, :]
            ri = ri_ref[rds, :]
            zc = zcol[rsl]
            for cc in range(QCHUNKS):
                c = q * QCHUNKS + cc
                sl = slice(c * LANES, (c + 1) * LANES)
                dj = (zc + sq[rsl, slice(cc * LANES, (cc + 1) * LANES)]
                      ) + cnb[None, sl]
                upd = dj < rv
                rv = jnp.where(upd, dj, rv)
                ri = jnp.where(upd, k * N_CHUNKS + c, ri)
            rv_ref[rds, :] = rv
            ri_ref[rds, :] = ri

    @pl.when(k == K_TILES - 1)
    def _():
        lane = lax.broadcasted_iota(jnp.int32, (BN, LANES), 1)
        rva = rv_ref[...]
        ria = ri_ref[...]
        tmin = jnp.min(rva, axis=1)
        idx_ref[...] = jnp.min(
            jnp.where(rva <= tmin[:, None], ria * LANES + lane, 2**30), axis=1)
        part = jnp.sum(tmin)

        @pl.when(n == 0)
        def _():
            acc_ref[0, 0] = part

        @pl.when(n > 0)
        def _():
            acc_ref[0, 0] = acc_ref[0, 0] + part

        @pl.when(n == N_TILES - 1)
        def _():
            loss_ref[0, 0] = acc_ref[0, 0] * (1.25 / (N_ROWS * D))


def _dist_argmin(zf, cbt, zn, cn):
    return pl.pallas_call(
        _dist_argmin_body,
        grid=(N_TILES, K_TILES),
        in_specs=[
            pl.BlockSpec((BN, D), lambda n, k: (n, 0)),
            pl.BlockSpec((D, BK), lambda n, k: (0, k)),
            pl.BlockSpec((BN,), lambda n, k: (n,)),
            pl.BlockSpec((BK,), lambda n, k: (k,)),
        ],
        out_specs=[
            pl.BlockSpec((BN,), lambda n, k: (n,)),
            pl.BlockSpec(memory_space=pltpu.SMEM),
        ],
        out_shape=[
            jax.ShapeDtypeStruct((N_ROWS,), jnp.int32),
            jax.ShapeDtypeStruct((1, 1), jnp.float32),
        ],
        scratch_shapes=[
            pltpu.SMEM((1, 1), jnp.float32),
            pltpu.VMEM((BN, LANES), jnp.float32),
            pltpu.VMEM((BN, LANES), jnp.int32),
        ],
        compiler_params=pltpu.CompilerParams(
            dimension_semantics=("arbitrary", "arbitrary")),
    )(zf, cbt, zn, cn)


@functools.cache
def _make_sc_gather():
    info = plsc.get_sparse_core_info()
    nc, ns = info.num_cores, info.num_subcores
    nw = nc * ns
    b_per_w = N_ROWS // nw
    n_chunks = b_per_w // 128
    mesh = plsc.VectorSubcoreMesh(core_axis_name="c", subcore_axis_name="s")

    @functools.partial(
        pl.kernel, mesh=mesh,
        out_type=jax.ShapeDtypeStruct((N_ROWS, D), jnp.float32),
        scratch_types=[
            pltpu.VMEM((n_chunks, 128), jnp.int32),
            pltpu.VMEM((b_per_w, D), jnp.float32),
            pltpu.SemaphoreType.DMA,
        ],
    )
    def gather(idx_hbm, table_hbm, out_hbm, idx_v, rows_v, sem):
        wid = lax.axis_index("s") * nc + lax.axis_index("c")
        base = wid * n_chunks
        pltpu.sync_copy(idx_hbm.at[pl.ds(base, n_chunks)], idx_v)
        handles = [
            pltpu.async_copy(table_hbm.at[idx_v.at[j]],
                             rows_v.at[pl.ds(j * 128, 128)], sem)
            for j in range(n_chunks)
        ]
        for h in handles:
            h.wait()
        pltpu.sync_copy(rows_v, out_hbm.at[pl.ds(wid * b_per_w, b_per_w)])

    return gather


def kernel(x, codebook):
    zf = jnp.transpose(x, (0, 2, 3, 1)).reshape(N_ROWS, D)
    zn = jnp.sum(zf ** 2, axis=1)
    cn = jnp.sum(codebook ** 2, axis=1)
    idx, loss2 = _dist_argmin(zf, codebook.T * -2.0, zn, cn)
    quant = _make_sc_gather()(idx.reshape(N_ROWS // 128, 128), codebook)
    out = quant.reshape(8, 32, 32, D).transpose(0, 3, 1, 2)
    return out, loss2[0, 0], idx


# speedup vs baseline: 1.2958x; 1.0465x over previous
import functools

import jax
import jax.numpy as jnp
from jax import lax
from jax.experimental import pallas as pl
from jax.experimental.pallas import tpu as pltpu
from jax.experimental.pallas import tpu_sc as plsc

N_ROWS = 8192
D = 256
K = 8192
BN = 1024
BK = 8192
LANES = 128
RG = 64
QCHUNKS = 4
N_TILES = N_ROWS // BN
K_TILES = K // BK
N_CHUNKS = BK // LANES


def _dist_argmin_body(z_ref, cbt_ref, zn_ref, cn_ref, idx_ref, loss_ref,
                      acc_ref, rv_ref, ri_ref):
    n = pl.program_id(0)
    k = pl.program_id(1)
    zb = z_ref[...]
    cbt = cbt_ref[...]
    znb = zn_ref[...]
    cnb = cn_ref[...]
    zcol = znb[:, None]

    @pl.when(k == 0)
    def _():
        rv_ref[...] = jnp.full((BN, LANES), jnp.inf, jnp.float32)
        ri_ref[...] = jnp.zeros((BN, LANES), jnp.int32)

    nq = N_CHUNKS // QCHUNKS
    for q in range(nq):
        qsl = slice(q * QCHUNKS * LANES, (q + 1) * QCHUNKS * LANES)
        sq = lax.dot_general(zb, cbt[:, qsl], (((1,), (0,)), ((), ())),
                             preferred_element_type=jnp.float32)
        for rg in range(BN // RG):
            rsl = slice(rg * RG, (rg + 1) * RG)
            rds = pl.ds(rg * RG, RG)
            rv = rv_ref[rds, :]
            ri = ri_ref[rds, :]
            zc = zcol[rsl]
            for cc in range(QCHUNKS):
                c = q * QCHUNKS + cc
                sl = slice(c * LANES, (c + 1) * LANES)
                dj = (zc + sq[rsl, slice(cc * LANES, (cc + 1) * LANES)]
                      ) + cnb[None, sl]
                upd = dj < rv
                rv = jnp.where(upd, dj, rv)
                ri = jnp.where(upd, k * N_CHUNKS + c, ri)
            rv_ref[rds, :] = rv
            ri_ref[rds, :] = ri

    @pl.when(k == K_TILES - 1)
    def _():
        lane = lax.broadcasted_iota(jnp.int32, (BN, LANES), 1)
        rva = rv_ref[...]
        ria = ri_ref[...]
        tmin = jnp.min(rva, axis=1)
        idx_ref[...] = jnp.min(
            jnp.where(rva <= tmin[:, None], ria * LANES + lane, 2**30), axis=1)
        part = jnp.sum(tmin)

        @pl.when(n == 0)
        def _():
            acc_ref[0, 0] = part

        @pl.when(n > 0)
        def _():
            acc_ref[0, 0] = acc_ref[0, 0] + part

        @pl.when(n == N_TILES - 1)
        def _():
            loss_ref[0, 0] = acc_ref[0, 0] * (1.25 / (N_ROWS * D))


def _dist_argmin(zf, cbt, zn, cn):
    return pl.pallas_call(
        _dist_argmin_body,
        grid=(N_TILES, K_TILES),
        in_specs=[
            pl.BlockSpec((BN, D), lambda n, k: (n, 0)),
            pl.BlockSpec((D, BK), lambda n, k: (0, k)),
            pl.BlockSpec((BN,), lambda n, k: (n,)),
            pl.BlockSpec((BK,), lambda n, k: (k,)),
        ],
        out_specs=[
            pl.BlockSpec((BN,), lambda n, k: (n,)),
            pl.BlockSpec(memory_space=pltpu.SMEM),
        ],
        out_shape=[
            jax.ShapeDtypeStruct((N_ROWS,), jnp.int32),
            jax.ShapeDtypeStruct((1, 1), jnp.float32),
        ],
        scratch_shapes=[
            pltpu.SMEM((1, 1), jnp.float32),
            pltpu.VMEM((BN, LANES), jnp.float32),
            pltpu.VMEM((BN, LANES), jnp.int32),
        ],
        compiler_params=pltpu.CompilerParams(
            dimension_semantics=("arbitrary", "arbitrary")),
    )(zf, cbt, zn, cn)


@functools.cache
def _make_sc_gather():
    info = plsc.get_sparse_core_info()
    nc, ns = info.num_cores, info.num_subcores
    nw = nc * ns
    b_per_w = N_ROWS // nw
    n_chunks = b_per_w // 128
    mesh = plsc.VectorSubcoreMesh(core_axis_name="c", subcore_axis_name="s")

    @functools.partial(
        pl.kernel, mesh=mesh,
        out_type=jax.ShapeDtypeStruct((N_ROWS, D), jnp.float32),
        scratch_types=[
            pltpu.VMEM((n_chunks, 128), jnp.int32),
            pltpu.VMEM((b_per_w, D), jnp.float32),
            pltpu.SemaphoreType.DMA,
        ],
    )
    def gather(idx_hbm, table_hbm, out_hbm, idx_v, rows_v, sem):
        wid = lax.axis_index("s") * nc + lax.axis_index("c")
        base = wid * n_chunks
        pltpu.sync_copy(idx_hbm.at[pl.ds(base, n_chunks)], idx_v)
        handles = [
            pltpu.async_copy(table_hbm.at[idx_v.at[j]],
                             rows_v.at[pl.ds(j * 128, 128)], sem)
            for j in range(n_chunks)
        ]
        for h in handles:
            h.wait()
        pltpu.sync_copy(rows_v, out_hbm.at[pl.ds(wid * b_per_w, b_per_w)])

    return gather


def kernel(x, codebook):
    zf = jnp.transpose(x, (0, 2, 3, 1)).reshape(N_ROWS, D)
    zn = jnp.sum(zf ** 2, axis=1)
    cn = jnp.sum(codebook ** 2, axis=1)
    idx, loss2 = _dist_argmin(zf, codebook.T * -2.0, zn, cn)
    quant = _make_sc_gather()(idx.reshape(N_ROWS // 128, 128), codebook)
    out = quant.reshape(8, 32, 32, D).transpose(0, 3, 1, 2)
    return out, loss2[0, 0], idx


# speedup vs baseline: 1.4755x; 1.1387x over previous
import functools

import jax
import jax.numpy as jnp
from jax import lax
from jax.experimental import pallas as pl
from jax.experimental.pallas import tpu as pltpu
from jax.experimental.pallas import tpu_sc as plsc

N_ROWS = 8192
D = 256
K = 8192
BN = 1024
BK = 8192
LANES = 128
RG = 64
QCHUNKS = 4
N_TILES = N_ROWS // BN
K_TILES = K // BK
N_CHUNKS = BK // LANES


def _dist_argmin_body(z_ref, cb_ref, zn_ref, cn_ref, idx_ref, loss_ref,
                      acc_ref, rv_ref, ri_ref):
    n = pl.program_id(0)
    k = pl.program_id(1)
    zb = z_ref[...] * -2.0
    cb = cb_ref[...]
    znb = zn_ref[...]
    cnb = cn_ref[...]
    zcol = znb[:, None]

    @pl.when(k == 0)
    def _():
        rv_ref[...] = jnp.full((BN, LANES), jnp.inf, jnp.float32)
        ri_ref[...] = jnp.zeros((BN, LANES), jnp.int32)

    nq = N_CHUNKS // QCHUNKS
    for q in range(nq):
        qsl = slice(q * QCHUNKS * LANES, (q + 1) * QCHUNKS * LANES)
        sq = lax.dot_general(zb, cb[qsl, :], (((1,), (1,)), ((), ())),
                             preferred_element_type=jnp.float32)
        for rg in range(BN // RG):
            rsl = slice(rg * RG, (rg + 1) * RG)
            rds = pl.ds(rg * RG, RG)
            rv = rv_ref[rds, :]
            ri = ri_ref[rds, :]
            zc = zcol[rsl]
            for cc in range(QCHUNKS):
                c = q * QCHUNKS + cc
                sl = slice(c * LANES, (c + 1) * LANES)
                dj = (zc + sq[rsl, slice(cc * LANES, (cc + 1) * LANES)]
                      ) + cnb[None, sl]
                upd = dj < rv
                rv = jnp.where(upd, dj, rv)
                ri = jnp.where(upd, k * N_CHUNKS + c, ri)
            rv_ref[rds, :] = rv
            ri_ref[rds, :] = ri

    @pl.when(k == K_TILES - 1)
    def _():
        lane = lax.broadcasted_iota(jnp.int32, (BN, LANES), 1)
        rva = rv_ref[...]
        ria = ri_ref[...]
        tmin = jnp.min(rva, axis=1)
        idx_ref[...] = jnp.min(
            jnp.where(rva <= tmin[:, None], ria * LANES + lane, 2**30), axis=1)
        part = jnp.sum(tmin)

        @pl.when(n == 0)
        def _():
            acc_ref[0, 0] = part

        @pl.when(n > 0)
        def _():
            acc_ref[0, 0] = acc_ref[0, 0] + part

        @pl.when(n == N_TILES - 1)
        def _():
            loss_ref[0, 0] = acc_ref[0, 0] * (1.25 / (N_ROWS * D))


def _dist_argmin(zf, cb, zn, cn):
    return pl.pallas_call(
        _dist_argmin_body,
        grid=(N_TILES, K_TILES),
        in_specs=[
            pl.BlockSpec((BN, D), lambda n, k: (n, 0)),
            pl.BlockSpec((BK, D), lambda n, k: (k, 0)),
            pl.BlockSpec((BN,), lambda n, k: (n,)),
            pl.BlockSpec((BK,), lambda n, k: (k,)),
        ],
        out_specs=[
            pl.BlockSpec((BN,), lambda n, k: (n,)),
            pl.BlockSpec(memory_space=pltpu.SMEM),
        ],
        out_shape=[
            jax.ShapeDtypeStruct((N_ROWS,), jnp.int32),
            jax.ShapeDtypeStruct((1, 1), jnp.float32),
        ],
        scratch_shapes=[
            pltpu.SMEM((1, 1), jnp.float32),
            pltpu.VMEM((BN, LANES), jnp.float32),
            pltpu.VMEM((BN, LANES), jnp.int32),
        ],
        compiler_params=pltpu.CompilerParams(
            dimension_semantics=("arbitrary", "arbitrary")),
    )(zf, cb, zn, cn)


@functools.cache
def _make_sc_gather():
    info = plsc.get_sparse_core_info()
    nc, ns = info.num_cores, info.num_subcores
    nw = nc * ns
    b_per_w = N_ROWS // nw
    n_chunks = b_per_w // 128
    mesh = plsc.VectorSubcoreMesh(core_axis_name="c", subcore_axis_name="s")

    @functools.partial(
        pl.kernel, mesh=mesh,
        out_type=jax.ShapeDtypeStruct((N_ROWS, D), jnp.float32),
        scratch_types=[
            pltpu.VMEM((n_chunks, 128), jnp.int32),
            pltpu.VMEM((b_per_w, D), jnp.float32),
            pltpu.SemaphoreType.DMA,
        ],
    )
    def gather(idx_hbm, table_hbm, out_hbm, idx_v, rows_v, sem):
        wid = lax.axis_index("s") * nc + lax.axis_index("c")
        base = wid * n_chunks
        pltpu.sync_copy(idx_hbm.at[pl.ds(base, n_chunks)], idx_v)
        handles = [
            pltpu.async_copy(table_hbm.at[idx_v.at[j]],
                             rows_v.at[pl.ds(j * 128, 128)], sem)
            for j in range(n_chunks)
        ]
        for h in handles:
            h.wait()
        pltpu.sync_copy(rows_v, out_hbm.at[pl.ds(wid * b_per_w, b_per_w)])

    return gather


def kernel(x, codebook):
    zf = jnp.transpose(x, (0, 2, 3, 1)).reshape(N_ROWS, D)
    zn = jnp.sum(zf ** 2, axis=1)
    cn = jnp.sum(codebook ** 2, axis=1)
    idx, loss2 = _dist_argmin(zf, codebook, zn, cn)
    quant = _make_sc_gather()(idx.reshape(N_ROWS // 128, 128), codebook)
    out = quant.reshape(8, 32, 32, D).transpose(0, 3, 1, 2)
    return out, loss2[0, 0], idx
